# baseline (speedup 1.0000x reference)
import functools

import jax
import jax.numpy as jnp
from jax import lax
from jax.experimental import pallas as pl
from jax.experimental.pallas import tpu as pltpu
from jax.experimental.pallas import tpu_sc as plsc

_VOCAB = 100000
_EMBED = 64
_HIDDEN = 128
_CTX = 200

_NC = 2
_NS = 16
_NW = _NC * _NS
_IDX_PER_W = 8
_ACTIVE_W = _CTX // _IDX_PER_W

_TILE = 2000
_GRID = _VOCAB // _TILE


def _sc_gather_sum(idx_hbm, emb_hbm, out_hbm, idx_v, rows_v, acc_v, sem):
    wid = lax.axis_index("s") * _NC + lax.axis_index("c")

    @pl.when(wid < _ACTIVE_W)
    def _gather():
        base = pl.multiple_of(wid * _IDX_PER_W, _IDX_PER_W)
        pltpu.sync_copy(idx_hbm.at[pl.ds(base, _IDX_PER_W)], idx_v)
        pltpu.async_copy(emb_hbm.at[idx_v], rows_v, sem).wait()
        for c in range(_EMBED // 16):
            acc = rows_v[0, pl.ds(c * 16, 16)]
            for j in range(1, _IDX_PER_W):
                acc = acc + rows_v[j, pl.ds(c * 16, 16)]
            acc_v[0, pl.ds(c * 16, 16)] = acc

    @pl.when(wid >= _ACTIVE_W)
    def _zero():
        for c in range(_EMBED // 16):
            acc_v[0, pl.ds(c * 16, 16)] = jnp.zeros((16,), jnp.float32)

    pltpu.sync_copy(acc_v, out_hbm.at[pl.ds(wid, 1)])


@functools.cache
def _sc_gather():
    return pl.kernel(
        _sc_gather_sum,
        out_type=jax.ShapeDtypeStruct((_NW, _EMBED), jnp.float32),
        mesh=plsc.VectorSubcoreMesh(core_axis_name="c", subcore_axis_name="s"),
        scratch_types=[
            pltpu.VMEM((_IDX_PER_W,), jnp.int32),
            pltpu.VMEM((_IDX_PER_W, _EMBED), jnp.float32),
            pltpu.VMEM((1, _EMBED), jnp.float32),
            pltpu.SemaphoreType.DMA,
        ],
        compiler_params=pltpu.CompilerParams(use_tc_tiling_on_sc=False),
    )


def _tc_mlp_softmax(parts_ref, w1_ref, b1_ref, w2_ref, b2_ref, out_ref, hid_ref):
    i = pl.program_id(0)

    @pl.when(i == 0)
    def _head():
        x = jnp.sum(parts_ref[...], axis=0, keepdims=True)
        h = lax.dot_general(
            x, w1_ref[...], (((1,), (1,)), ((), ())),
            preferred_element_type=jnp.float32,
        ) + b1_ref[...]
        hid_ref[...] = jnp.maximum(h, 0.0)

    logits = lax.dot_general(
        hid_ref[...], w2_ref[...], (((1,), (1,)), ((), ())),
        preferred_element_type=jnp.float32,
    ) + b2_ref[0]
    out_ref[i] = logits

    @pl.when(i == _GRID - 1)
    def _softmax():
        full = out_ref[...]
        m = jnp.max(full)
        lse = m + jnp.log(jnp.sum(jnp.exp(full - m)))
        out_ref[...] = full - lse


_tc_call = pl.pallas_call(
    _tc_mlp_softmax,
    grid=(_GRID,),
    in_specs=[
        pl.BlockSpec((_NW, _EMBED), lambda i: (0, 0)),
        pl.BlockSpec((_HIDDEN, _EMBED), lambda i: (0, 0)),
        pl.BlockSpec((1, _HIDDEN), lambda i: (0, 0)),
        pl.BlockSpec((_TILE, _HIDDEN), lambda i: (i, 0)),
        pl.BlockSpec((1, 1, _TILE), lambda i: (i, 0, 0)),
    ],
    out_specs=pl.BlockSpec((_GRID, 1, _TILE), lambda i: (0, 0, 0)),
    out_shape=jax.ShapeDtypeStruct((_GRID, 1, _TILE), jnp.float32),
    scratch_shapes=[pltpu.VMEM((1, _HIDDEN), jnp.float32)],
    compiler_params=pltpu.CompilerParams(
        dimension_semantics=("arbitrary",),
    ),
)


@jax.jit
def kernel(inputs, emb, W1, b1, W2, b2):
    parts = _sc_gather()(inputs, emb)
    out3 = _tc_call(
        parts,
        W1,
        b1.reshape(1, _HIDDEN),
        W2,
        b2.reshape(_GRID, 1, _TILE),
    )
    return out3.reshape(1, _VOCAB)
